# baseline (speedup 1.0000x reference)
import functools

import jax
import jax.numpy as jnp
from jax import lax
from jax.experimental import pallas as pl
from jax.experimental.pallas import tpu as pltpu
from jax.experimental.pallas import tpu_sc as plsc

S, D = 2048, 1024
NH, NKV, HD = 16, 8, 64
E, FFN = 8, 3584
EPS = 1e-5
THETA = 10000.0
BT = 256
FT = 512
NTB = S // BT
NFT = FFN // FT
HALF = HD // 2
NEG = float(jnp.finfo(jnp.float32).min)


def _qkv_body(x_ref, ln1_ref, wqkv_ref, cos_ref, sin_ref, q_ref, k_ref, v_ref):
    x = x_ref[...]
    var = jnp.mean(x * x, axis=-1, keepdims=True)
    h = (x * jax.lax.rsqrt(var + EPS) * ln1_ref[...]).astype(jnp.bfloat16)
    qkv = jnp.dot(h, wqkv_ref[...], preferred_element_type=jnp.float32)
    cos = cos_ref[...][:, None, :]
    sin = sin_ref[...][:, None, :]

    def rope(z):
        z1 = z[..., :HALF]
        z2 = z[..., HALF:]
        return jnp.concatenate([z1 * cos - z2 * sin, z2 * cos + z1 * sin], axis=-1)

    q = rope(qkv[:, : NH * HD].reshape(BT, NH, HD))
    k = rope(qkv[:, NH * HD : (NH + NKV) * HD].reshape(BT, NKV, HD))
    v = qkv[:, (NH + NKV) * HD :].reshape(BT, NKV, HD)
    q_ref[...] = q.transpose(1, 0, 2).astype(jnp.bfloat16)
    k_ref[...] = k.transpose(1, 0, 2).astype(jnp.bfloat16)
    v_ref[...] = v.transpose(1, 0, 2).astype(jnp.bfloat16)


def _attn_body(q_ref, k_ref, v_ref, o_ref):
    i = pl.program_id(1)
    q = q_ref[0]
    s = jax.lax.dot_general(q, k_ref[0], (((1,), (1,)), ((), ())),
                            preferred_element_type=jnp.float32)
    s = s * (HD ** -0.5)
    row = i * BT + jax.lax.broadcasted_iota(jnp.int32, (BT, S), 0)
    col = jax.lax.broadcasted_iota(jnp.int32, (BT, S), 1)
    s = jnp.where(row >= col, s, NEG)
    m = jnp.max(s, axis=-1, keepdims=True)
    p = jnp.exp(s - m)
    l = jnp.sum(p, axis=-1, keepdims=True)
    a = (p / l).astype(jnp.bfloat16)
    o_ref[0] = jnp.dot(a, v_ref[0], preferred_element_type=jnp.float32).astype(jnp.bfloat16)


def _ores_body(ao_ref, wo_ref, x_ref, ln2_ref, wg_ref, t_ref, sel_ref, tw_ref):
    ao = ao_ref[...].transpose(1, 0, 2).reshape(BT, NH * HD)
    o = jnp.dot(ao, wo_ref[...], preferred_element_type=jnp.float32)
    r = o + x_ref[...]
    var = jnp.mean(r * r, axis=-1, keepdims=True)
    t = r * jax.lax.rsqrt(var + EPS) * ln2_ref[...]
    tb = t.astype(jnp.bfloat16)
    t_ref[...] = tb
    logits = jnp.dot(tb, wg_ref[...], preferred_element_type=jnp.float32)
    m = jnp.max(logits, axis=-1, keepdims=True)
    ex = jnp.exp(logits - m)
    rw = ex / jnp.sum(ex, axis=-1, keepdims=True)
    idx = jax.lax.broadcasted_iota(jnp.int32, (BT, E), 1)
    m0 = jnp.max(rw, axis=-1, keepdims=True)
    i0 = jnp.min(jnp.where(rw == m0, idx, E), axis=-1, keepdims=True)
    rw1 = jnp.where(idx == i0, -1.0, rw)
    m1 = jnp.max(rw1, axis=-1, keepdims=True)
    i1 = jnp.min(jnp.where(rw1 == m1, idx, E), axis=-1, keepdims=True)
    sw = m0 + m1
    sel_ref[...] = jnp.concatenate([i0, i1], axis=1)
    tw_ref[...] = jnp.concatenate([m0 / sw, m1 / sw], axis=1)


BR = 128
NP = 2 * S + E * BR
NBLK = NP // BR
FT2 = 1792
NFT2 = FFN // FT2


def _plan_body(sel_ref, pos_ref, be_ref, nbu_ref):
    sel = sel_ref[...]
    ide = jax.lax.broadcasted_iota(jnp.int32, (S, E), 1)
    oh0 = (sel[:, 0:1] == ide).astype(jnp.float32)
    oh1 = (sel[:, 1:2] == ide).astype(jnp.float32)
    oh = oh0 + oh1
    tri = (jax.lax.broadcasted_iota(jnp.int32, (256, 256), 0)
           > jax.lax.broadcasted_iota(jnp.int32, (256, 256), 1)).astype(jnp.bfloat16)
    chunks = []
    carry = jnp.zeros((1, E), jnp.float32)
    for c in range(S // 256):
        blk = oh[c * 256:(c + 1) * 256]
        cs = jnp.dot(tri, blk.astype(jnp.bfloat16), preferred_element_type=jnp.float32)
        chunks.append(cs + carry)
        carry = carry + jnp.sum(blk, axis=0, keepdims=True)
    C = jnp.concatenate(chunks, axis=0)
    counts = carry
    nb = jnp.floor((counts + (BR - 1)) / BR)
    triu8 = (jax.lax.broadcasted_iota(jnp.int32, (E, E), 0)
             <= jax.lax.broadcasted_iota(jnp.int32, (E, E), 1)).astype(jnp.bfloat16)
    incl = jnp.dot(nb.astype(jnp.bfloat16), triu8, preferred_element_type=jnp.float32)
    poff = (incl - nb) * BR
    pos0 = jnp.sum(oh0 * (poff + C), axis=-1, keepdims=True)
    pos1 = jnp.sum(oh1 * (poff + C), axis=-1, keepdims=True)
    pos_ref[...] = jnp.concatenate([pos0, pos1], axis=1).astype(jnp.int32)
    bi = jax.lax.broadcasted_iota(jnp.int32, (1, 64), 1).astype(jnp.float32)
    be_raw = jnp.sum((bi >= incl.reshape(E, 1)).astype(jnp.float32), axis=0, keepdims=True)
    be_ref[...] = jnp.minimum(be_raw, float(E - 1)).astype(jnp.int32)
    nbu_ref[...] = incl[0:1, E - 1:E].astype(jnp.int32)


def _gmm_a_body(be_ref, nbu_ref, xs_ref, w1_ref, w3_ref, h_ref):
    i = pl.program_id(1)

    @pl.when(i < nbu_ref[0])
    def _():
        x = xs_ref[pl.ds(i * BR, BR), :]
        a = jnp.dot(x, w1_ref[0].astype(jnp.bfloat16), preferred_element_type=jnp.float32)
        a = jax.nn.silu(a)
        b = jnp.dot(x, w3_ref[0].astype(jnp.bfloat16), preferred_element_type=jnp.float32)
        h_ref[...] = (a * b).astype(jnp.bfloat16)


def _gmm_b_body(be_ref, nbu_ref, h_ref, w2_ref, o_ref):
    i = pl.program_id(0)

    @pl.when(i < nbu_ref[0])
    def _():
        o_ref[...] = jnp.dot(h_ref[...], w2_ref[0].astype(jnp.bfloat16),
                             preferred_element_type=jnp.float32)


_SC_NC = 2
_SC_NS = 16
_SC_NW = _SC_NC * _SC_NS
_RPW = S // _SC_NW


def _disp_body(x_hbm, p0_hbm, p1_hbm, xs_hbm, idx_v, rows_v, sem):
    wid = lax.axis_index("s") * _SC_NC + lax.axis_index("c")
    base = wid * _RPW
    pltpu.sync_copy(x_hbm.at[pl.ds(base, _RPW)], rows_v)
    pltpu.sync_copy(p0_hbm.at[pl.ds(base, _RPW)], idx_v)
    pltpu.async_copy(rows_v, xs_hbm.at[idx_v], sem).wait()
    pltpu.sync_copy(p1_hbm.at[pl.ds(base, _RPW)], idx_v)
    pltpu.async_copy(rows_v, xs_hbm.at[idx_v], sem).wait()


def _gath_body(o_hbm, p0_hbm, p1_hbm, g0_hbm, g1_hbm, idx_v, rows_v, sem):
    wid = lax.axis_index("s") * _SC_NC + lax.axis_index("c")
    base = wid * _RPW
    pltpu.sync_copy(p0_hbm.at[pl.ds(base, _RPW)], idx_v)
    pltpu.async_copy(o_hbm.at[idx_v], rows_v, sem).wait()
    pltpu.sync_copy(rows_v, g0_hbm.at[pl.ds(base, _RPW)])
    pltpu.sync_copy(p1_hbm.at[pl.ds(base, _RPW)], idx_v)
    pltpu.async_copy(o_hbm.at[idx_v], rows_v, sem).wait()
    pltpu.sync_copy(rows_v, g1_hbm.at[pl.ds(base, _RPW)])


def _combine_body(g0_ref, g1_ref, tw_ref, out_ref):
    tw = tw_ref[...]
    out_ref[...] = tw[:, 0:1] * g0_ref[...] + tw[:, 1:2] * g1_ref[...]


def kernel(hidden_states, positions, ln1_w, wq, wk, wv, wo, ln2_w, wg, w1, w2, w3):
    x = hidden_states.reshape(S, D)
    inv_freq = 1.0 / (THETA ** (jnp.arange(HALF, dtype=jnp.float32) * 2.0 / HD))
    ang = positions.reshape(S).astype(jnp.float32)[:, None] * inv_freq[None, :]
    cos = jnp.cos(ang)
    sin = jnp.sin(ang)
    wqkv = jnp.concatenate([wq, wk, wv], axis=1).astype(jnp.bfloat16)

    q, k, v = pl.pallas_call(
        _qkv_body,
        grid=(NTB,),
        in_specs=[
            pl.BlockSpec((BT, D), lambda i: (i, 0)),
            pl.BlockSpec((1, D), lambda i: (0, 0)),
            pl.BlockSpec((D, (NH + 2 * NKV) * HD), lambda i: (0, 0)),
            pl.BlockSpec((BT, HALF), lambda i: (i, 0)),
            pl.BlockSpec((BT, HALF), lambda i: (i, 0)),
        ],
        out_specs=[
            pl.BlockSpec((NH, BT, HD), lambda i: (0, i, 0)),
            pl.BlockSpec((NKV, BT, HD), lambda i: (0, i, 0)),
            pl.BlockSpec((NKV, BT, HD), lambda i: (0, i, 0)),
        ],
        out_shape=[
            jax.ShapeDtypeStruct((NH, S, HD), jnp.bfloat16),
            jax.ShapeDtypeStruct((NKV, S, HD), jnp.bfloat16),
            jax.ShapeDtypeStruct((NKV, S, HD), jnp.bfloat16),
        ],
    )(x, ln1_w.reshape(1, D), wqkv, cos, sin)

    ao = pl.pallas_call(
        _attn_body,
        grid=(NH, NTB),
        in_specs=[
            pl.BlockSpec((1, BT, HD), lambda h, i: (h, i, 0)),
            pl.BlockSpec((1, S, HD), lambda h, i: (h // 2, 0, 0)),
            pl.BlockSpec((1, S, HD), lambda h, i: (h // 2, 0, 0)),
        ],
        out_specs=pl.BlockSpec((1, BT, HD), lambda h, i: (h, i, 0)),
        out_shape=jax.ShapeDtypeStruct((NH, S, HD), jnp.bfloat16),
    )(q, k, v)

    t, sel, tw = pl.pallas_call(
        _ores_body,
        grid=(NTB,),
        in_specs=[
            pl.BlockSpec((NH, BT, HD), lambda i: (0, i, 0)),
            pl.BlockSpec((NH * HD, D), lambda i: (0, 0)),
            pl.BlockSpec((BT, D), lambda i: (i, 0)),
            pl.BlockSpec((1, D), lambda i: (0, 0)),
            pl.BlockSpec((D, E), lambda i: (0, 0)),
        ],
        out_specs=[
            pl.BlockSpec((BT, D), lambda i: (i, 0)),
            pl.BlockSpec((BT, 2), lambda i: (i, 0)),
            pl.BlockSpec((BT, 2), lambda i: (i, 0)),
        ],
        out_shape=[
            jax.ShapeDtypeStruct((S, D), jnp.bfloat16),
            jax.ShapeDtypeStruct((S, 2), jnp.int32),
            jax.ShapeDtypeStruct((S, 2), jnp.float32),
        ],
    )(ao, wo.astype(jnp.bfloat16), x, ln2_w.reshape(1, D), wg.astype(jnp.bfloat16))

    pos, be, nbu = pl.pallas_call(
        _plan_body,
        out_shape=[
            jax.ShapeDtypeStruct((S, 2), jnp.int32),
            jax.ShapeDtypeStruct((1, 64), jnp.int32),
            jax.ShapeDtypeStruct((1, 1), jnp.int32),
        ],
    )(sel)

    pos0 = pos[:, 0]
    pos1 = pos[:, 1]
    t32 = lax.bitcast_convert_type(t.reshape(S, D // 2, 2), jnp.float32)
    xs32 = pl.kernel(
        _disp_body,
        mesh=plsc.VectorSubcoreMesh(core_axis_name="c", subcore_axis_name="s"),
        out_type=jax.ShapeDtypeStruct((NP, D // 2), jnp.float32),
        scratch_types=[
            pltpu.VMEM((_RPW,), jnp.int32),
            pltpu.VMEM((_RPW, D // 2), jnp.float32),
            pltpu.SemaphoreType.DMA,
        ],
    )(t32, pos0, pos1)
    xs = lax.bitcast_convert_type(xs32, jnp.bfloat16).reshape(NP, D)

    h = pl.pallas_call(
        _gmm_a_body,
        grid_spec=pltpu.PrefetchScalarGridSpec(
            num_scalar_prefetch=2,
            grid=(NFT2, NBLK),
            in_specs=[
                pl.BlockSpec((NP, D), lambda f, i, be_r, nbu_r: (0, 0)),
                pl.BlockSpec((1, D, FT2), lambda f, i, be_r, nbu_r: (be_r[i], 0, f)),
                pl.BlockSpec((1, D, FT2), lambda f, i, be_r, nbu_r: (be_r[i], 0, f)),
            ],
            out_specs=pl.BlockSpec((BR, FT2), lambda f, i, be_r, nbu_r: (i, f)),
        ),
        out_shape=jax.ShapeDtypeStruct((NP, FFN), jnp.bfloat16),
    )(be.reshape(64), nbu.reshape(1), xs, w1, w3)

    o = pl.pallas_call(
        _gmm_b_body,
        grid_spec=pltpu.PrefetchScalarGridSpec(
            num_scalar_prefetch=2,
            grid=(NBLK,),
            in_specs=[
                pl.BlockSpec((BR, FFN), lambda i, be_r, nbu_r: (i, 0)),
                pl.BlockSpec((1, FFN, D), lambda i, be_r, nbu_r: (be_r[i], 0, 0)),
            ],
            out_specs=pl.BlockSpec((BR, D), lambda i, be_r, nbu_r: (i, 0)),
        ),
        out_shape=jax.ShapeDtypeStruct((NP, D), jnp.float32),
    )(be.reshape(64), nbu.reshape(1), h, w2)

    g0, g1 = pl.kernel(
        _gath_body,
        mesh=plsc.VectorSubcoreMesh(core_axis_name="c", subcore_axis_name="s"),
        out_type=[
            jax.ShapeDtypeStruct((S, D), jnp.float32),
            jax.ShapeDtypeStruct((S, D), jnp.float32),
        ],
        scratch_types=[
            pltpu.VMEM((_RPW,), jnp.int32),
            pltpu.VMEM((_RPW, D), jnp.float32),
            pltpu.SemaphoreType.DMA,
        ],
    )(o, pos0, pos1)

    out = pl.pallas_call(
        _combine_body,
        grid=(NTB,),
        in_specs=[
            pl.BlockSpec((BT, D), lambda i: (i, 0)),
            pl.BlockSpec((BT, D), lambda i: (i, 0)),
            pl.BlockSpec((BT, 2), lambda i: (i, 0)),
        ],
        out_specs=pl.BlockSpec((BT, D), lambda i: (i, 0)),
        out_shape=jax.ShapeDtypeStruct((S, D), jnp.float32),
    )(g0, g1, tw)

    return out.reshape(1, S, D)


# speedup vs baseline: 1.2665x; 1.2665x over previous
import functools

import jax
import jax.numpy as jnp
from jax import lax
from jax.experimental import pallas as pl
from jax.experimental.pallas import tpu as pltpu
from jax.experimental.pallas import tpu_sc as plsc

S, D = 2048, 1024
NH, NKV, HD = 16, 8, 64
E, FFN = 8, 3584
EPS = 1e-5
THETA = 10000.0
BT = 256
FT = 512
NTB = S // BT
NFT = FFN // FT
HALF = HD // 2
NEG = float(jnp.finfo(jnp.float32).min)


def _qkv_body(x_ref, ln1_ref, wqkv_ref, cos_ref, sin_ref, q_ref, k_ref, v_ref):
    x = x_ref[...]
    var = jnp.mean(x * x, axis=-1, keepdims=True)
    h = (x * jax.lax.rsqrt(var + EPS) * ln1_ref[...]).astype(jnp.bfloat16)
    qkv = jnp.dot(h, wqkv_ref[...], preferred_element_type=jnp.float32)
    cos = cos_ref[...][:, None, :]
    sin = sin_ref[...][:, None, :]

    def rope(z):
        z1 = z[..., :HALF]
        z2 = z[..., HALF:]
        return jnp.concatenate([z1 * cos - z2 * sin, z2 * cos + z1 * sin], axis=-1)

    q = rope(qkv[:, : NH * HD].reshape(BT, NH, HD))
    k = rope(qkv[:, NH * HD : (NH + NKV) * HD].reshape(BT, NKV, HD))
    v = qkv[:, (NH + NKV) * HD :].reshape(BT, NKV, HD)
    q_ref[...] = q.transpose(1, 0, 2).astype(jnp.bfloat16)
    k_ref[...] = k.transpose(1, 0, 2).astype(jnp.bfloat16)
    v_ref[...] = v.transpose(1, 0, 2).astype(jnp.bfloat16)


def _attn_body(q_ref, k_ref, v_ref, o_ref):
    i = pl.program_id(1)
    q = q_ref[0]
    s = jax.lax.dot_general(q, k_ref[0], (((1,), (1,)), ((), ())),
                            preferred_element_type=jnp.float32)
    s = s * (HD ** -0.5)
    row = i * BT + jax.lax.broadcasted_iota(jnp.int32, (BT, S), 0)
    col = jax.lax.broadcasted_iota(jnp.int32, (BT, S), 1)
    s = jnp.where(row >= col, s, NEG)
    m = jnp.max(s, axis=-1, keepdims=True)
    p = jnp.exp(s - m)
    l = jnp.sum(p, axis=-1, keepdims=True)
    a = (p / l).astype(jnp.bfloat16)
    o_ref[0] = jnp.dot(a, v_ref[0], preferred_element_type=jnp.float32).astype(jnp.bfloat16)


def _ores_body(ao_ref, wo_ref, x_ref, ln2_ref, wg_ref, t_ref, sel_ref, tw_ref):
    ao = ao_ref[...].transpose(1, 0, 2).reshape(BT, NH * HD)
    o = jnp.dot(ao, wo_ref[...], preferred_element_type=jnp.float32)
    r = o + x_ref[...]
    var = jnp.mean(r * r, axis=-1, keepdims=True)
    t = r * jax.lax.rsqrt(var + EPS) * ln2_ref[...]
    t_ref[...] = t
    logits = jnp.dot(t.astype(jnp.bfloat16), wg_ref[...],
                     preferred_element_type=jnp.float32)
    m = jnp.max(logits, axis=-1, keepdims=True)
    ex = jnp.exp(logits - m)
    rw = ex / jnp.sum(ex, axis=-1, keepdims=True)
    idx = jax.lax.broadcasted_iota(jnp.int32, (BT, E), 1)
    m0 = jnp.max(rw, axis=-1, keepdims=True)
    i0 = jnp.min(jnp.where(rw == m0, idx, E), axis=-1, keepdims=True)
    rw1 = jnp.where(idx == i0, -1.0, rw)
    m1 = jnp.max(rw1, axis=-1, keepdims=True)
    i1 = jnp.min(jnp.where(rw1 == m1, idx, E), axis=-1, keepdims=True)
    sw = m0 + m1
    sel_ref[...] = jnp.concatenate([i0, i1], axis=1)
    tw_ref[...] = jnp.concatenate([m0 / sw, m1 / sw], axis=1)


BR = 128
NP = 2 * S + E * BR
NBLK = NP // BR
FT2 = 1792
NFT2 = FFN // FT2


def _plan_body(sel_ref, pos_ref, be_ref, nbu_ref):
    sel = sel_ref[...]
    ide = jax.lax.broadcasted_iota(jnp.int32, (S, E), 1)
    oh0 = (sel[:, 0:1] == ide).astype(jnp.float32)
    oh1 = (sel[:, 1:2] == ide).astype(jnp.float32)
    oh = oh0 + oh1
    tri = (jax.lax.broadcasted_iota(jnp.int32, (256, 256), 0)
           > jax.lax.broadcasted_iota(jnp.int32, (256, 256), 1)).astype(jnp.bfloat16)
    chunks = []
    carry = jnp.zeros((1, E), jnp.float32)
    for c in range(S // 256):
        blk = oh[c * 256:(c + 1) * 256]
        cs = jnp.dot(tri, blk.astype(jnp.bfloat16), preferred_element_type=jnp.float32)
        chunks.append(cs + carry)
        carry = carry + jnp.sum(blk, axis=0, keepdims=True)
    C = jnp.concatenate(chunks, axis=0)
    counts = carry
    nb = jnp.floor((counts + (BR - 1)) / BR)
    triu8 = (jax.lax.broadcasted_iota(jnp.int32, (E, E), 0)
             <= jax.lax.broadcasted_iota(jnp.int32, (E, E), 1)).astype(jnp.bfloat16)
    incl = jnp.dot(nb.astype(jnp.bfloat16), triu8, preferred_element_type=jnp.float32)
    poff = (incl - nb) * BR
    pos0 = jnp.sum(oh0 * (poff + C), axis=-1, keepdims=True)
    pos1 = jnp.sum(oh1 * (poff + C), axis=-1, keepdims=True)
    pos_ref[...] = jnp.concatenate([pos0, pos1], axis=1).astype(jnp.int32)
    bi = jax.lax.broadcasted_iota(jnp.int32, (1, 64), 1).astype(jnp.float32)
    be_raw = jnp.sum((bi >= incl.reshape(E, 1)).astype(jnp.float32), axis=0, keepdims=True)
    be_ref[...] = jnp.minimum(be_raw, float(E - 1)).astype(jnp.int32)
    nbu_ref[...] = incl[0:1, E - 1:E].astype(jnp.int32)


def _gmm_a_body(be_ref, nbu_ref, xs_ref, w1_ref, w3_ref, h_ref):
    i = pl.program_id(1)

    @pl.when(i < nbu_ref[0])
    def _():
        x = xs_ref[...].astype(jnp.bfloat16)
        a = jnp.dot(x, w1_ref[0].astype(jnp.bfloat16), preferred_element_type=jnp.float32)
        a = jax.nn.silu(a)
        b = jnp.dot(x, w3_ref[0].astype(jnp.bfloat16), preferred_element_type=jnp.float32)
        h_ref[...] = (a * b).astype(jnp.bfloat16)


def _gmm_b_body(be_ref, nbu_ref, h_ref, w2_ref, o_ref):
    i = pl.program_id(0)

    @pl.when(i < nbu_ref[0])
    def _():
        o_ref[...] = jnp.dot(h_ref[...], w2_ref[0].astype(jnp.bfloat16),
                             preferred_element_type=jnp.float32)


_SC_NC = 2
_SC_NS = 16
_SC_NW = _SC_NC * _SC_NS
_RPW = S // _SC_NW


def _disp_body(x_hbm, p0_hbm, p1_hbm, xs_hbm, idx_v, rows_v, sem):
    wid = lax.axis_index("s") * _SC_NC + lax.axis_index("c")
    base = wid * _RPW
    pltpu.sync_copy(x_hbm.at[pl.ds(base, _RPW)], rows_v)
    pltpu.sync_copy(p0_hbm.at[pl.ds(base, _RPW)], idx_v)
    pltpu.async_copy(rows_v, xs_hbm.at[idx_v], sem).wait()
    pltpu.sync_copy(p1_hbm.at[pl.ds(base, _RPW)], idx_v)
    pltpu.async_copy(rows_v, xs_hbm.at[idx_v], sem).wait()


def _gath_body(o_hbm, p0_hbm, p1_hbm, g0_hbm, g1_hbm, idx_v, rows_v, sem):
    wid = lax.axis_index("s") * _SC_NC + lax.axis_index("c")
    base = wid * _RPW
    pltpu.sync_copy(p0_hbm.at[pl.ds(base, _RPW)], idx_v)
    pltpu.async_copy(o_hbm.at[idx_v], rows_v, sem).wait()
    pltpu.sync_copy(rows_v, g0_hbm.at[pl.ds(base, _RPW)])
    pltpu.sync_copy(p1_hbm.at[pl.ds(base, _RPW)], idx_v)
    pltpu.async_copy(o_hbm.at[idx_v], rows_v, sem).wait()
    pltpu.sync_copy(rows_v, g1_hbm.at[pl.ds(base, _RPW)])


def _combine_body(g0_ref, g1_ref, tw_ref, out_ref):
    tw = tw_ref[...]
    out_ref[...] = tw[:, 0:1] * g0_ref[...] + tw[:, 1:2] * g1_ref[...]


def kernel(hidden_states, positions, ln1_w, wq, wk, wv, wo, ln2_w, wg, w1, w2, w3):
    x = hidden_states.reshape(S, D)
    inv_freq = 1.0 / (THETA ** (jnp.arange(HALF, dtype=jnp.float32) * 2.0 / HD))
    ang = positions.reshape(S).astype(jnp.float32)[:, None] * inv_freq[None, :]
    cos = jnp.cos(ang)
    sin = jnp.sin(ang)
    wqkv = jnp.concatenate([wq, wk, wv], axis=1).astype(jnp.bfloat16)

    q, k, v = pl.pallas_call(
        _qkv_body,
        grid=(NTB,),
        in_specs=[
            pl.BlockSpec((BT, D), lambda i: (i, 0)),
            pl.BlockSpec((1, D), lambda i: (0, 0)),
            pl.BlockSpec((D, (NH + 2 * NKV) * HD), lambda i: (0, 0)),
            pl.BlockSpec((BT, HALF), lambda i: (i, 0)),
            pl.BlockSpec((BT, HALF), lambda i: (i, 0)),
        ],
        out_specs=[
            pl.BlockSpec((NH, BT, HD), lambda i: (0, i, 0)),
            pl.BlockSpec((NKV, BT, HD), lambda i: (0, i, 0)),
            pl.BlockSpec((NKV, BT, HD), lambda i: (0, i, 0)),
        ],
        out_shape=[
            jax.ShapeDtypeStruct((NH, S, HD), jnp.bfloat16),
            jax.ShapeDtypeStruct((NKV, S, HD), jnp.bfloat16),
            jax.ShapeDtypeStruct((NKV, S, HD), jnp.bfloat16),
        ],
    )(x, ln1_w.reshape(1, D), wqkv, cos, sin)

    ao = pl.pallas_call(
        _attn_body,
        grid=(NH, NTB),
        in_specs=[
            pl.BlockSpec((1, BT, HD), lambda h, i: (h, i, 0)),
            pl.BlockSpec((1, S, HD), lambda h, i: (h // 2, 0, 0)),
            pl.BlockSpec((1, S, HD), lambda h, i: (h // 2, 0, 0)),
        ],
        out_specs=pl.BlockSpec((1, BT, HD), lambda h, i: (h, i, 0)),
        out_shape=jax.ShapeDtypeStruct((NH, S, HD), jnp.bfloat16),
    )(q, k, v)

    t, sel, tw = pl.pallas_call(
        _ores_body,
        grid=(NTB,),
        in_specs=[
            pl.BlockSpec((NH, BT, HD), lambda i: (0, i, 0)),
            pl.BlockSpec((NH * HD, D), lambda i: (0, 0)),
            pl.BlockSpec((BT, D), lambda i: (i, 0)),
            pl.BlockSpec((1, D), lambda i: (0, 0)),
            pl.BlockSpec((D, E), lambda i: (0, 0)),
        ],
        out_specs=[
            pl.BlockSpec((BT, D), lambda i: (i, 0)),
            pl.BlockSpec((BT, 2), lambda i: (i, 0)),
            pl.BlockSpec((BT, 2), lambda i: (i, 0)),
        ],
        out_shape=[
            jax.ShapeDtypeStruct((S, D), jnp.float32),
            jax.ShapeDtypeStruct((S, 2), jnp.int32),
            jax.ShapeDtypeStruct((S, 2), jnp.float32),
        ],
    )(ao, wo.astype(jnp.bfloat16), x, ln2_w.reshape(1, D), wg.astype(jnp.bfloat16))

    pos, be, nbu = pl.pallas_call(
        _plan_body,
        out_shape=[
            jax.ShapeDtypeStruct((S, 2), jnp.int32),
            jax.ShapeDtypeStruct((1, 64), jnp.int32),
            jax.ShapeDtypeStruct((1, 1), jnp.int32),
        ],
    )(sel)

    pos0 = pos[:, 0]
    pos1 = pos[:, 1]
    xs = pl.kernel(
        _disp_body,
        mesh=plsc.VectorSubcoreMesh(core_axis_name="c", subcore_axis_name="s"),
        out_type=jax.ShapeDtypeStruct((NP, D), jnp.float32),
        scratch_types=[
            pltpu.VMEM((_RPW,), jnp.int32),
            pltpu.VMEM((_RPW, D), jnp.float32),
            pltpu.SemaphoreType.DMA,
        ],
    )(t, pos0, pos1)

    h = pl.pallas_call(
        _gmm_a_body,
        grid_spec=pltpu.PrefetchScalarGridSpec(
            num_scalar_prefetch=2,
            grid=(NFT2, NBLK),
            in_specs=[
                pl.BlockSpec((BR, D), lambda f, i, be_r, nbu_r: (i, 0)),
                pl.BlockSpec((1, D, FT2), lambda f, i, be_r, nbu_r: (be_r[i], 0, f)),
                pl.BlockSpec((1, D, FT2), lambda f, i, be_r, nbu_r: (be_r[i], 0, f)),
            ],
            out_specs=pl.BlockSpec((BR, FT2), lambda f, i, be_r, nbu_r: (i, f)),
        ),
        out_shape=jax.ShapeDtypeStruct((NP, FFN), jnp.bfloat16),
    )(be.reshape(64), nbu.reshape(1), xs, w1, w3)

    o = pl.pallas_call(
        _gmm_b_body,
        grid_spec=pltpu.PrefetchScalarGridSpec(
            num_scalar_prefetch=2,
            grid=(NBLK,),
            in_specs=[
                pl.BlockSpec((BR, FFN), lambda i, be_r, nbu_r: (i, 0)),
                pl.BlockSpec((1, FFN, D), lambda i, be_r, nbu_r: (be_r[i], 0, 0)),
            ],
            out_specs=pl.BlockSpec((BR, D), lambda i, be_r, nbu_r: (i, 0)),
        ),
        out_shape=jax.ShapeDtypeStruct((NP, D), jnp.float32),
    )(be.reshape(64), nbu.reshape(1), h, w2)

    g0, g1 = pl.kernel(
        _gath_body,
        mesh=plsc.VectorSubcoreMesh(core_axis_name="c", subcore_axis_name="s"),
        out_type=[
            jax.ShapeDtypeStruct((S, D), jnp.float32),
            jax.ShapeDtypeStruct((S, D), jnp.float32),
        ],
        scratch_types=[
            pltpu.VMEM((_RPW,), jnp.int32),
            pltpu.VMEM((_RPW, D), jnp.float32),
            pltpu.SemaphoreType.DMA,
        ],
    )(o, pos0, pos1)

    out = pl.pallas_call(
        _combine_body,
        grid=(NTB,),
        in_specs=[
            pl.BlockSpec((BT, D), lambda i: (i, 0)),
            pl.BlockSpec((BT, D), lambda i: (i, 0)),
            pl.BlockSpec((BT, 2), lambda i: (i, 0)),
        ],
        out_specs=pl.BlockSpec((BT, D), lambda i: (i, 0)),
        out_shape=jax.ShapeDtypeStruct((S, D), jnp.float32),
    )(g0, g1, tw)

    return out.reshape(1, S, D)
